# baseline (speedup 1.0000x reference)
import functools

import jax
import jax.numpy as jnp
from jax import lax
from jax.experimental import pallas as pl
from jax.experimental.pallas import tpu as pltpu
from jax.experimental.pallas import tpu_sc as plsc

Q = 512
D = 256
N = 65536
K = 16
BK = 4096
NB = N // BK
NCH = 32
CH = BK // NCH


def _merge_into_running(rv, ri, cv, ci):
    mv = jnp.concatenate([rv, cv], axis=1)
    mi = jnp.concatenate([ri, ci], axis=1)
    nv, ni = [], []
    for _ in range(K):
        m = jnp.min(mv, axis=1, keepdims=True)
        ci_min = jnp.min(jnp.where(mv == m, mi, jnp.int32(2 * N)),
                         axis=1, keepdims=True)
        mask = (mv == m) & (mi == ci_min)
        mv = jnp.where(mask, jnp.float32(jnp.inf), mv)
        nv.append(m)
        ni.append(ci_min)
    return jnp.concatenate(nv, axis=1), jnp.concatenate(ni, axis=1)


def _topk_body(qref, kref, od_ref, oi_ref, rv_ref, ri_ref, s3_ref, flag_ref):
    j = pl.program_id(0)

    @pl.when(j == 0)
    def _init():
        rv_ref[...] = jnp.full((Q, K), jnp.inf, dtype=jnp.float32)
        ri_ref[...] = jnp.zeros((Q, K), dtype=jnp.int32)

    q = qref[...]
    kb = kref[...]
    q_sq = jnp.sum(q * q, axis=1, keepdims=True)
    k_sq = jnp.sum(kb * kb, axis=1)[None, :]
    mm = lax.dot_general(q, kb, (((1,), (1,)), ((), ())),
                         preferred_element_type=jnp.float32)
    d2 = q_sq + k_sq - 2.0 * mm
    dist = jnp.sqrt(jnp.maximum(d2, 1e-12))
    s3_ref[...] = dist.reshape(Q, NCH, CH)
    flag_ref[0] = 1

    base = j * BK
    iota3 = lax.broadcasted_iota(jnp.int32, (Q, NCH, CH), 2)
    chunk_base = base + CH * lax.broadcasted_iota(jnp.int32, (Q, NCH), 1)

    for _ in range(K):
        @pl.when(flag_ref[0] == 1)
        def _round():
            s3 = s3_ref[...]
            mc = jnp.min(s3, axis=2)
            bb = jnp.min(mc, axis=1, keepdims=True)
            t_cur = rv_ref[:, K - 1:K]
            need = jnp.any(bb <= t_cur)

            @pl.when(need)
            def _extract():
                cand = jnp.where(s3 == mc[:, :, None], iota3, jnp.int32(CH))
                ixc = jnp.min(cand, axis=2)
                s3_ref[...] = jnp.where(iota3 == ixc[:, :, None],
                                        jnp.float32(jnp.inf), s3)
                gidx = chunk_base + ixc
                new_v, new_i = _merge_into_running(
                    rv_ref[...], ri_ref[...], mc, gidx)
                rv_ref[...] = new_v
                ri_ref[...] = new_i

            @pl.when(jnp.logical_not(need))
            def _stop():
                flag_ref[0] = 0

    @pl.when(j == NB - 1)
    def _done():
        od_ref[...] = rv_ref[...]
        oi_ref[...] = ri_ref[...]


def _topk_call(queries, keys, interpret=False):
    return pl.pallas_call(
        _topk_body,
        grid=(NB,),
        in_specs=[
            pl.BlockSpec((Q, D), lambda j: (0, 0)),
            pl.BlockSpec((BK, D), lambda j: (j, 0)),
        ],
        out_specs=[
            pl.BlockSpec((Q, K), lambda j: (0, 0)),
            pl.BlockSpec((Q, K), lambda j: (0, 0)),
        ],
        out_shape=[
            jax.ShapeDtypeStruct((Q, K), jnp.float32),
            jax.ShapeDtypeStruct((Q, K), jnp.int32),
        ],
        scratch_shapes=[
            pltpu.VMEM((Q, K), jnp.float32),
            pltpu.VMEM((Q, K), jnp.int32),
            pltpu.VMEM((Q, NCH, CH), jnp.float32),
            pltpu.SMEM((1,), jnp.int32),
        ],
        compiler_params=pltpu.CompilerParams(
            dimension_semantics=("arbitrary",),
        ),
        interpret=interpret,
    )(queries, keys)


def _make_sc_gather():
    info = plsc.get_sparse_core_info()
    nw = info.num_cores * info.num_subcores
    b_per_w = Q // nw
    mesh = plsc.VectorSubcoreMesh(core_axis_name="c", subcore_axis_name="s")

    @functools.partial(
        pl.kernel,
        mesh=mesh,
        out_type=jax.ShapeDtypeStruct((Q, D), jnp.float32),
        scratch_types=[
            pltpu.VMEM((b_per_w,), jnp.int32),
            pltpu.VMEM((b_per_w, D), jnp.float32),
            pltpu.SemaphoreType.DMA,
        ],
    )
    def _gather(table_hbm, idx_hbm, out_hbm, idx_v, rows_v, sem):
        wid = lax.axis_index("s") * info.num_cores + lax.axis_index("c")
        base = wid * b_per_w
        pltpu.sync_copy(idx_hbm.at[pl.ds(base, b_per_w)], idx_v)
        pltpu.async_copy(table_hbm.at[idx_v], rows_v, sem).wait()
        pltpu.sync_copy(rows_v, out_hbm.at[pl.ds(base, b_per_w)])

    return _gather


def kernel(queries, keys, k):
    del k
    topk_dists, topk_idx = _topk_call(queries, keys)
    nearest = _make_sc_gather()(keys, topk_idx[:, 0])
    return (topk_dists, topk_idx, nearest)


# speedup vs baseline: 3.0477x; 3.0477x over previous
import functools

import jax
import jax.numpy as jnp
from jax import lax
from jax.experimental import pallas as pl
from jax.experimental.pallas import tpu as pltpu
from jax.experimental.pallas import tpu_sc as plsc

Q = 512
D = 256
N = 65536
K = 16
BK = 4096
NB = N // BK
NCH = 32
CH = BK // NCH


def _merge_into_running(rv, ri, cv, ci):
    mv = jnp.concatenate([rv, cv], axis=1)
    mi = jnp.concatenate([ri, ci], axis=1)
    nv, ni = [], []
    for _ in range(K):
        m = jnp.min(mv, axis=1, keepdims=True)
        ci_min = jnp.min(jnp.where(mv == m, mi, jnp.int32(2 * N)),
                         axis=1, keepdims=True)
        mask = (mv == m) & (mi == ci_min)
        mv = jnp.where(mask, jnp.float32(jnp.inf), mv)
        nv.append(m)
        ni.append(ci_min)
    return jnp.concatenate(nv, axis=1), jnp.concatenate(ni, axis=1)


def _topk_body(qref, kref, od_ref, oi_ref, rv_ref, ri_ref, s_ref):
    j = pl.program_id(0)

    @pl.when(j == 0)
    def _init():
        rv_ref[...] = jnp.full((Q, K), jnp.inf, dtype=jnp.float32)
        ri_ref[...] = jnp.zeros((Q, K), dtype=jnp.int32)

    q = qref[...]
    kb = kref[...]
    q_sq = jnp.sum(q * q, axis=1, keepdims=True)
    k_sq = jnp.sum(kb * kb, axis=1)[None, :]
    mm = lax.dot_general(q, kb, (((1,), (1,)), ((), ())),
                         preferred_element_type=jnp.float32)
    d2 = q_sq + k_sq - 2.0 * mm
    dist = jnp.sqrt(jnp.maximum(d2, 1e-12))
    s_ref[...] = dist

    base = j * BK
    iota = lax.broadcasted_iota(jnp.int32, (Q, CH), 1)

    mc0 = jnp.concatenate(
        [jnp.min(dist[:, c * CH:(c + 1) * CH], axis=1, keepdims=True)
         for c in range(NCH)], axis=1)
    bb0 = jnp.min(mc0, axis=1, keepdims=True)
    need0 = jnp.any(bb0 <= rv_ref[:, K - 1:K])

    def _round(need):
        del need
        nmcs, gidxs = [], []
        for c in range(NCH):
            sl = s_ref[:, c * CH:(c + 1) * CH]
            mc_c = jnp.min(sl, axis=1, keepdims=True)
            cand = jnp.where(sl == mc_c, iota, jnp.int32(CH))
            ix_c = jnp.min(cand, axis=1, keepdims=True)
            masked = jnp.where(iota == ix_c, jnp.float32(jnp.inf), sl)
            s_ref[:, c * CH:(c + 1) * CH] = masked
            nmcs.append((jnp.min(masked, axis=1, keepdims=True), mc_c))
            gidxs.append(ix_c + (base + c * CH))
        mc = jnp.concatenate([p[1] for p in nmcs], axis=1)
        gidx = jnp.concatenate(gidxs, axis=1)
        new_v, new_i = _merge_into_running(
            rv_ref[...], ri_ref[...], mc, gidx)
        rv_ref[...] = new_v
        ri_ref[...] = new_i
        nmc = jnp.concatenate([p[0] for p in nmcs], axis=1)
        bb = jnp.min(nmc, axis=1, keepdims=True)
        return jnp.any(bb <= new_v[:, K - 1:K])

    lax.while_loop(lambda need: need, _round, need0)

    @pl.when(j == NB - 1)
    def _done():
        od_ref[...] = rv_ref[...]
        oi_ref[...] = ri_ref[...]


def _topk_call(queries, keys, interpret=False):
    return pl.pallas_call(
        _topk_body,
        grid=(NB,),
        in_specs=[
            pl.BlockSpec((Q, D), lambda j: (0, 0)),
            pl.BlockSpec((BK, D), lambda j: (j, 0)),
        ],
        out_specs=[
            pl.BlockSpec((Q, K), lambda j: (0, 0)),
            pl.BlockSpec((Q, K), lambda j: (0, 0)),
        ],
        out_shape=[
            jax.ShapeDtypeStruct((Q, K), jnp.float32),
            jax.ShapeDtypeStruct((Q, K), jnp.int32),
        ],
        scratch_shapes=[
            pltpu.VMEM((Q, K), jnp.float32),
            pltpu.VMEM((Q, K), jnp.int32),
            pltpu.VMEM((Q, BK), jnp.float32),
        ],
        compiler_params=pltpu.CompilerParams(
            dimension_semantics=("arbitrary",),
        ),
        interpret=interpret,
    )(queries, keys)


def _make_sc_gather():
    info = plsc.get_sparse_core_info()
    nw = info.num_cores * info.num_subcores
    b_per_w = Q // nw
    mesh = plsc.VectorSubcoreMesh(core_axis_name="c", subcore_axis_name="s")

    @functools.partial(
        pl.kernel,
        mesh=mesh,
        out_type=jax.ShapeDtypeStruct((Q, D), jnp.float32),
        scratch_types=[
            pltpu.VMEM((b_per_w,), jnp.int32),
            pltpu.VMEM((b_per_w, D), jnp.float32),
            pltpu.SemaphoreType.DMA,
        ],
    )
    def _gather(table_hbm, idx_hbm, out_hbm, idx_v, rows_v, sem):
        wid = lax.axis_index("s") * info.num_cores + lax.axis_index("c")
        base = wid * b_per_w
        pltpu.sync_copy(idx_hbm.at[pl.ds(base, b_per_w)], idx_v)
        pltpu.async_copy(table_hbm.at[idx_v], rows_v, sem).wait()
        pltpu.sync_copy(rows_v, out_hbm.at[pl.ds(base, b_per_w)])

    return _gather


def kernel(queries, keys, k):
    del k
    topk_dists, topk_idx = _topk_call(queries, keys)
    nearest = _make_sc_gather()(keys, topk_idx[:, 0])
    return (topk_dists, topk_idx, nearest)
